# Initial kernel scaffold; baseline (speedup 1.0000x reference)
import jax
import jax.numpy as jnp
from jax.experimental import pallas as pl


def kernel(logits, labels, router_logits):
    raise NotImplementedError("write your pallas kernel here")


# speedup vs baseline: 1.0931x; 1.0931x over previous
import functools

import jax
import jax.numpy as jnp
from jax.experimental import pallas as pl
from jax.experimental.pallas import tpu as pltpu

_NUM_EXPERTS = 8
_TOP_K = 2
_AUX_COEF = 0.02
_IGNORE = -100

_R = 256
_VB = 3200
_V = 32000
_N = 2048
_L = 32
_E = 8


def _ce_body(x_ref, lab_ref, nll_ref, cnt_ref, m_ref, s_ref, g_ref):
    i = pl.program_id(0)
    j = pl.program_id(1)
    nvb = pl.num_programs(1)

    @pl.when(jnp.logical_and(i == 0, j == 0))
    def _init_out():
        nll_ref[:, :] = jnp.zeros_like(nll_ref)
        cnt_ref[:, :] = jnp.zeros_like(cnt_ref)

    @pl.when(j == 0)
    def _init_scratch():
        m_ref[:, :] = jnp.full((_R, 1), -jnp.inf, jnp.float32)
        s_ref[:, :] = jnp.zeros((_R, 1), jnp.float32)
        g_ref[:, :] = jnp.zeros((_R, 1), jnp.float32)

    x = x_ref[:, :]
    bm = jnp.max(x, axis=1, keepdims=True)
    m_old = m_ref[:, :]
    m_new = jnp.maximum(m_old, bm)
    s_ref[:, :] = (s_ref[:, :] * jnp.exp(m_old - m_new)
                   + jnp.sum(jnp.exp(x - m_new), axis=1, keepdims=True))
    m_ref[:, :] = m_new

    lab = lab_ref[0, 0, :].reshape(_R, 1)
    cols = jax.lax.broadcasted_iota(jnp.int32, (_R, _VB), 1) + j * _VB
    hit = cols == lab
    g_ref[:, :] += jnp.sum(jnp.where(hit, x, 0.0), axis=1, keepdims=True)

    @pl.when(j == nvb - 1)
    def _finish():
        valid = lab != _IGNORE
        nll = m_ref[:, :] + jnp.log(s_ref[:, :]) - g_ref[:, :]
        nll = jnp.where(valid, nll, 0.0)
        nll_ref[:, :] += jnp.sum(nll).reshape(1, 1)
        cnt_ref[:, :] += jnp.sum(valid.astype(jnp.float32)).reshape(1, 1)


def _aux_body(r_ref, lab_ref, cp_ref, den_ref, cnt_ref, pe_ref, ws_ref):
    l = pl.program_id(0)
    nl = pl.num_programs(0)

    @pl.when(l == 0)
    def _init():
        cnt_ref[:, :] = jnp.zeros((1, _E), jnp.float32)
        pe_ref[:, :] = jnp.zeros((1, _E), jnp.float32)
        ws_ref[:, :] = jnp.zeros((1, 1), jnp.float32)

    x = r_ref[0]
    w = (lab_ref[0, 0, :] != _IGNORE).astype(jnp.float32).reshape(_N, 1)

    mx = jnp.max(x, axis=1, keepdims=True)
    ex = jnp.exp(x - mx)
    prob = ex / jnp.sum(ex, axis=1, keepdims=True)

    jidx = jax.lax.broadcasted_iota(jnp.int32, (_N, _E), 1)
    ind_cols = []
    for e in range(_E):
        xe = x[:, e:e + 1]
        beats = jnp.logical_or(x > xe, jnp.logical_and(x == xe, jidx < e))
        rank = jnp.sum(beats.astype(jnp.float32), axis=1, keepdims=True)
        ind_cols.append((rank < _TOP_K).astype(jnp.float32))
    ind = jnp.concatenate(ind_cols, axis=1)

    cnt_ref[:, :] += jnp.sum(w * ind, axis=0, keepdims=True)
    pe_ref[:, :] += jnp.sum(w * prob, axis=0, keepdims=True)
    ws_ref[:, :] += jnp.sum(w).reshape(1, 1)

    @pl.when(l == nl - 1)
    def _finish():
        cp_ref[:, :] = jnp.sum(cnt_ref[:, :] * pe_ref[:, :]).reshape(1, 1)
        den_ref[:, :] = ws_ref[:, :]


def kernel(logits, labels, router_logits):
    n = logits.shape[1]
    v = logits.shape[-1]
    logits2 = logits.reshape(n, v)
    lab_flat = labels.reshape(-1)
    shift_lab = jnp.concatenate(
        [lab_flat[1:], jnp.full((1,), _IGNORE, jnp.int32)])
    shift_lab3 = shift_lab.reshape(n // _R, 1, _R)

    nll_sum, valid_cnt = pl.pallas_call(
        _ce_body,
        grid=(n // _R, v // _VB),
        in_specs=[
            pl.BlockSpec((_R, _VB), lambda i, j: (i, j)),
            pl.BlockSpec((1, 1, _R), lambda i, j: (i, 0, 0)),
        ],
        out_specs=[
            pl.BlockSpec((1, 1), lambda i, j: (0, 0)),
            pl.BlockSpec((1, 1), lambda i, j: (0, 0)),
        ],
        out_shape=[
            jax.ShapeDtypeStruct((1, 1), jnp.float32),
            jax.ShapeDtypeStruct((1, 1), jnp.float32),
        ],
        scratch_shapes=[
            pltpu.VMEM((_R, 1), jnp.float32),
            pltpu.VMEM((_R, 1), jnp.float32),
            pltpu.VMEM((_R, 1), jnp.float32),
        ],
        compiler_params=pltpu.CompilerParams(
            dimension_semantics=("arbitrary", "arbitrary")),
    )(logits2, shift_lab3)

    lab3 = lab_flat.reshape(1, 1, n)
    cp, den = pl.pallas_call(
        _aux_body,
        grid=(_L,),
        in_specs=[
            pl.BlockSpec((1, _N, _E), lambda l: (l, 0, 0)),
            pl.BlockSpec((1, 1, _N), lambda l: (0, 0, 0)),
        ],
        out_specs=[
            pl.BlockSpec((1, 1), lambda l: (0, 0)),
            pl.BlockSpec((1, 1), lambda l: (0, 0)),
        ],
        out_shape=[
            jax.ShapeDtypeStruct((1, 1), jnp.float32),
            jax.ShapeDtypeStruct((1, 1), jnp.float32),
        ],
        scratch_shapes=[
            pltpu.VMEM((1, _E), jnp.float32),
            pltpu.VMEM((1, _E), jnp.float32),
            pltpu.VMEM((1, 1), jnp.float32),
        ],
        compiler_params=pltpu.CompilerParams(
            dimension_semantics=("arbitrary",)),
    )(router_logits, lab3)

    loss = nll_sum[0, 0] / valid_cnt[0, 0]
    aux = _NUM_EXPERTS * cp[0, 0] / (den[0, 0] * den[0, 0])
    return loss + _AUX_COEF * aux


# speedup vs baseline: 1.1340x; 1.0374x over previous
import functools

import jax
from jax import lax
import jax.numpy as jnp
from jax.experimental import pallas as pl
from jax.experimental.pallas import tpu as pltpu
from jax.experimental.pallas import tpu_sc as plsc

_NUM_EXPERTS = 8
_TOP_K = 2
_AUX_COEF = 0.02
_IGNORE = -100

_R = 256
_VB = 3200
_V = 32000
_N = 2048
_L = 32
_E = 8

_NW = 32
_PW = _N // _NW
_SCL = 16


def _lse_body(x_ref, lab_ref, lse_ref, cnt_ref, m_ref, s_ref):
    i = pl.program_id(0)
    j = pl.program_id(1)
    nvb = pl.num_programs(1)

    @pl.when(jnp.logical_and(i == 0, j == 0))
    def _init_out():
        lse_ref[:, :] = jnp.zeros_like(lse_ref)
        cnt_ref[:, :] = jnp.zeros_like(cnt_ref)

    @pl.when(j == 0)
    def _init_scratch():
        m_ref[:, :] = jnp.full((_R, 1), -jnp.inf, jnp.float32)
        s_ref[:, :] = jnp.zeros((_R, 1), jnp.float32)

    x = x_ref[:, :]
    bm = jnp.max(x, axis=1, keepdims=True)
    m_old = m_ref[:, :]
    m_new = jnp.maximum(m_old, bm)
    s_ref[:, :] = (s_ref[:, :] * jnp.exp(m_old - m_new)
                   + jnp.sum(jnp.exp(x - m_new), axis=1, keepdims=True))
    m_ref[:, :] = m_new

    @pl.when(j == nvb - 1)
    def _finish():
        lab = lab_ref[0, 0, :].reshape(_R, 1)
        valid = lab != _IGNORE
        lse = m_ref[:, :] + jnp.log(s_ref[:, :])
        lse = jnp.where(valid, lse, 0.0)
        lse_ref[:, :] += jnp.sum(lse).reshape(1, 1)
        cnt_ref[:, :] += jnp.sum(valid.astype(jnp.float32)).reshape(1, 1)


def _gather_body(logits_hbm, lab_hbm, out_hbm, lab_v, idx_v, g_v, acc_v, sem):
    c = lax.axis_index("c")
    s = lax.axis_index("s")
    wid = s * 2 + c
    base = wid * _PW
    pltpu.sync_copy(lab_hbm.at[pl.ds(base, _PW)], lab_v)
    for ch in range(_PW // _SCL):
        lab = lab_v[pl.ds(ch * _SCL, _SCL)]
        pos = base + ch * _SCL + lax.iota(jnp.int32, _SCL)
        idx_v[pl.ds(ch * _SCL, _SCL)] = jnp.where(lab >= 0, pos * _V + lab, 0)
    pltpu.async_copy(logits_hbm.at[idx_v], g_v, sem).wait()
    acc = jnp.zeros((_SCL,), jnp.float32)
    for ch in range(_PW // _SCL):
        lab = lab_v[pl.ds(ch * _SCL, _SCL)]
        g = g_v[pl.ds(ch * _SCL, _SCL)]
        acc = acc + jnp.where(lab >= 0, g, 0.0)
    acc_v[...] = acc
    pltpu.sync_copy(acc_v, out_hbm.at[wid])


def _aux_body(r_ref, lab_ref, cp_ref, den_ref, cnt_ref, pe_ref, ws_ref):
    l = pl.program_id(0)
    nl = pl.num_programs(0)

    @pl.when(l == 0)
    def _init():
        cnt_ref[:, :] = jnp.zeros((_E, 128), jnp.float32)
        pe_ref[:, :] = jnp.zeros((_E, 128), jnp.float32)
        ws_ref[:, :] = jnp.zeros((1, 128), jnp.float32)

    x = r_ref[0]
    w = (lab_ref[:, :] != _IGNORE).astype(jnp.float32)

    mx = jnp.max(x, axis=0, keepdims=True)
    ex = jnp.exp(x - mx)
    prob = ex / jnp.sum(ex, axis=0, keepdims=True)

    sub = jax.lax.broadcasted_iota(jnp.int32, (_E, _N), 0)
    rank = jnp.zeros((_E, _N), jnp.float32)
    for j in range(_E):
        xj = x[j:j + 1, :]
        beats = jnp.logical_or(xj > x, jnp.logical_and(xj == x, j < sub))
        rank += beats.astype(jnp.float32)
    ind = (rank < _TOP_K).astype(jnp.float32)

    cnt_ref[:, :] += jnp.sum((ind * w).reshape(_E, _N // 128, 128), axis=1)
    pe_ref[:, :] += jnp.sum((prob * w).reshape(_E, _N // 128, 128), axis=1)
    ws_ref[:, :] += jnp.sum(w.reshape(1, _N // 128, 128), axis=1)

    @pl.when(l == nl - 1)
    def _finish():
        cnt = jnp.sum(cnt_ref[:, :], axis=1, keepdims=True)
        pe = jnp.sum(pe_ref[:, :], axis=1, keepdims=True)
        cp_ref[:, :] = jnp.sum(cnt * pe).reshape(1, 1)
        den_ref[:, :] = jnp.sum(ws_ref[:, :]).reshape(1, 1)


def kernel(logits, labels, router_logits):
    n = logits.shape[1]
    v = logits.shape[-1]
    logits2 = logits.reshape(n, v)
    lab_flat = labels.reshape(-1)
    shift_lab = jnp.concatenate(
        [lab_flat[1:], jnp.full((1,), _IGNORE, jnp.int32)])
    shift_lab3 = shift_lab.reshape(n // _R, 1, _R)

    sc_gather = functools.partial(
        pl.kernel,
        mesh=plsc.VectorSubcoreMesh(core_axis_name="c", subcore_axis_name="s"),
        out_type=jax.ShapeDtypeStruct((_NW, _SCL), jnp.float32),
        scratch_types=[
            pltpu.VMEM((_PW,), jnp.int32),
            pltpu.VMEM((_PW,), jnp.int32),
            pltpu.VMEM((_PW,), jnp.float32),
            pltpu.VMEM((_SCL,), jnp.float32),
            pltpu.SemaphoreType.DMA,
        ],
    )(_gather_body)
    g_partials = sc_gather(logits.reshape(-1), shift_lab)

    lse_sum, valid_cnt = pl.pallas_call(
        _lse_body,
        grid=(n // _R, v // _VB),
        in_specs=[
            pl.BlockSpec((_R, _VB), lambda i, j: (i, j)),
            pl.BlockSpec((1, 1, _R), lambda i, j: (i, 0, 0)),
        ],
        out_specs=[
            pl.BlockSpec((1, 1), lambda i, j: (0, 0)),
            pl.BlockSpec((1, 1), lambda i, j: (0, 0)),
        ],
        out_shape=[
            jax.ShapeDtypeStruct((1, 1), jnp.float32),
            jax.ShapeDtypeStruct((1, 1), jnp.float32),
        ],
        scratch_shapes=[
            pltpu.VMEM((_R, 1), jnp.float32),
            pltpu.VMEM((_R, 1), jnp.float32),
        ],
        compiler_params=pltpu.CompilerParams(
            dimension_semantics=("arbitrary", "arbitrary")),
    )(logits2, shift_lab3)

    lab2 = lab_flat.reshape(1, n)
    router_t = router_logits.transpose(0, 2, 1)
    cp, den = pl.pallas_call(
        _aux_body,
        grid=(_L,),
        in_specs=[
            pl.BlockSpec((1, _E, _N), lambda l: (l, 0, 0)),
            pl.BlockSpec((1, _N), lambda l: (0, 0)),
        ],
        out_specs=[
            pl.BlockSpec((1, 1), lambda l: (0, 0)),
            pl.BlockSpec((1, 1), lambda l: (0, 0)),
        ],
        out_shape=[
            jax.ShapeDtypeStruct((1, 1), jnp.float32),
            jax.ShapeDtypeStruct((1, 1), jnp.float32),
        ],
        scratch_shapes=[
            pltpu.VMEM((_E, 128), jnp.float32),
            pltpu.VMEM((_E, 128), jnp.float32),
            pltpu.VMEM((1, 128), jnp.float32),
        ],
        compiler_params=pltpu.CompilerParams(
            dimension_semantics=("arbitrary",)),
    )(router_t, lab2)

    g_sum = jnp.sum(g_partials)
    loss = (lse_sum[0, 0] - g_sum) / valid_cnt[0, 0]
    aux = _NUM_EXPERTS * cp[0, 0] / (den[0, 0] * den[0, 0])
    return loss + _AUX_COEF * aux


# speedup vs baseline: 1.1941x; 1.0529x over previous
import functools

import jax
from jax import lax
import jax.numpy as jnp
from jax.experimental import pallas as pl
from jax.experimental.pallas import tpu as pltpu
from jax.experimental.pallas import tpu_sc as plsc

_NUM_EXPERTS = 8
_TOP_K = 2
_AUX_COEF = 0.02
_IGNORE = -100

_R = 256
_VB = 3200
_NS = 2
_V = 32000
_N = 2048
_L = 32
_E = 8

_NW = 32
_PW = _N // _NW
_SCL = 16


def _lse_body(*refs):
    xs = [r[:, :] for r in refs[:_NS]]
    lab_ref, lse_ref, cnt_ref, m_ref, s_ref = refs[_NS:]
    i = pl.program_id(0)
    j = pl.program_id(1)
    nvb = pl.num_programs(1)

    @pl.when(jnp.logical_and(i == 0, j == 0))
    def _init_out():
        lse_ref[:, :] = jnp.zeros_like(lse_ref)
        cnt_ref[:, :] = jnp.zeros_like(cnt_ref)

    @pl.when(j == 0)
    def _init_scratch():
        m_ref[:, :] = jnp.full((_R, 1), -jnp.inf, jnp.float32)
        s_ref[:, :] = jnp.zeros((_R, 1), jnp.float32)

    bm = jnp.max(xs[0], axis=1, keepdims=True)
    for x in xs[1:]:
        bm = jnp.maximum(bm, jnp.max(x, axis=1, keepdims=True))
    m_old = m_ref[:, :]
    m_new = jnp.maximum(m_old, bm)
    acc = s_ref[:, :] * jnp.exp(m_old - m_new)
    for x in xs:
        acc = acc + jnp.sum(jnp.exp(x - m_new), axis=1, keepdims=True)
    s_ref[:, :] = acc
    m_ref[:, :] = m_new

    @pl.when(j == nvb - 1)
    def _finish():
        lab = lab_ref[0, 0, :].reshape(_R, 1)
        valid = lab != _IGNORE
        lse = m_ref[:, :] + jnp.log(s_ref[:, :])
        lse = jnp.where(valid, lse, 0.0)
        lse_ref[:, :] += jnp.sum(lse).reshape(1, 1)
        cnt_ref[:, :] += jnp.sum(valid.astype(jnp.float32)).reshape(1, 1)


def _gather_body(logits_hbm, lab_hbm, out_hbm, lab_v, idx_v, g_v, acc_v, sem):
    c = lax.axis_index("c")
    s = lax.axis_index("s")
    wid = s * 2 + c
    base = wid * _PW
    pltpu.sync_copy(lab_hbm.at[pl.ds(base, _PW)], lab_v)
    for ch in range(_PW // _SCL):
        lab = lab_v[pl.ds(ch * _SCL, _SCL)]
        pos = base + ch * _SCL + lax.iota(jnp.int32, _SCL)
        idx_v[pl.ds(ch * _SCL, _SCL)] = jnp.where(lab >= 0, pos * _V + lab, 0)
    pltpu.async_copy(logits_hbm.at[idx_v], g_v, sem).wait()
    acc = jnp.zeros((_SCL,), jnp.float32)
    for ch in range(_PW // _SCL):
        lab = lab_v[pl.ds(ch * _SCL, _SCL)]
        g = g_v[pl.ds(ch * _SCL, _SCL)]
        acc = acc + jnp.where(lab >= 0, g, 0.0)
    acc_v[...] = acc
    pltpu.sync_copy(acc_v, out_hbm.at[wid])


def _aux_body(r_ref, lab_ref, cp_ref, den_ref, cnt_ref, pe_ref, ws_ref):
    l = pl.program_id(0)
    nl = pl.num_programs(0)

    @pl.when(l == 0)
    def _init():
        cnt_ref[:, :] = jnp.zeros((_E, 128), jnp.float32)
        pe_ref[:, :] = jnp.zeros((_E, 128), jnp.float32)
        ws_ref[:, :] = jnp.zeros((1, 128), jnp.float32)

    x = r_ref[0]
    w = (lab_ref[:, :] != _IGNORE).astype(jnp.float32)

    mx = jnp.max(x, axis=0, keepdims=True)
    ex = jnp.exp(x - mx)
    prob = ex / jnp.sum(ex, axis=0, keepdims=True)

    sub = jax.lax.broadcasted_iota(jnp.int32, (_E, _N), 0)
    rank = jnp.zeros((_E, _N), jnp.float32)
    for j in range(_E):
        xj = x[j:j + 1, :]
        beats = jnp.logical_or(xj > x, jnp.logical_and(xj == x, j < sub))
        rank += beats.astype(jnp.float32)
    ind = (rank < _TOP_K).astype(jnp.float32)

    cnt_ref[:, :] += jnp.sum((ind * w).reshape(_E, _N // 128, 128), axis=1)
    pe_ref[:, :] += jnp.sum((prob * w).reshape(_E, _N // 128, 128), axis=1)
    ws_ref[:, :] += jnp.sum(w.reshape(1, _N // 128, 128), axis=1)

    @pl.when(l == nl - 1)
    def _finish():
        cnt = jnp.sum(cnt_ref[:, :], axis=1, keepdims=True)
        pe = jnp.sum(pe_ref[:, :], axis=1, keepdims=True)
        cp_ref[:, :] = jnp.sum(cnt * pe).reshape(1, 1)
        den_ref[:, :] = jnp.sum(ws_ref[:, :]).reshape(1, 1)


def kernel(logits, labels, router_logits):
    n = logits.shape[1]
    v = logits.shape[-1]
    logits2 = logits.reshape(n, v)
    lab_flat = labels.reshape(-1)
    shift_lab = jnp.concatenate(
        [lab_flat[1:], jnp.full((1,), _IGNORE, jnp.int32)])
    shift_lab3 = shift_lab.reshape(n // _R, 1, _R)

    sc_gather = functools.partial(
        pl.kernel,
        mesh=plsc.VectorSubcoreMesh(core_axis_name="c", subcore_axis_name="s"),
        out_type=jax.ShapeDtypeStruct((_NW, _SCL), jnp.float32),
        scratch_types=[
            pltpu.VMEM((_PW,), jnp.int32),
            pltpu.VMEM((_PW,), jnp.int32),
            pltpu.VMEM((_PW,), jnp.float32),
            pltpu.VMEM((_SCL,), jnp.float32),
            pltpu.SemaphoreType.DMA,
        ],
    )(_gather_body)
    g_partials = sc_gather(logits.reshape(-1), shift_lab)

    lse_sum, valid_cnt = pl.pallas_call(
        _lse_body,
        grid=(n // _R, v // (_NS * _VB)),
        in_specs=[
            pl.BlockSpec((_R, _VB),
                         functools.partial(
                             lambda k, i, j: (i, _NS * j + k), k))
            for k in range(_NS)
        ] + [
            pl.BlockSpec((1, 1, _R), lambda i, j: (i, 0, 0)),
        ],
        out_specs=[
            pl.BlockSpec((1, 1), lambda i, j: (0, 0)),
            pl.BlockSpec((1, 1), lambda i, j: (0, 0)),
        ],
        out_shape=[
            jax.ShapeDtypeStruct((1, 1), jnp.float32),
            jax.ShapeDtypeStruct((1, 1), jnp.float32),
        ],
        scratch_shapes=[
            pltpu.VMEM((_R, 1), jnp.float32),
            pltpu.VMEM((_R, 1), jnp.float32),
        ],
        compiler_params=pltpu.CompilerParams(
            dimension_semantics=("arbitrary", "arbitrary")),
    )(*([logits2] * _NS), shift_lab3)

    lab2 = lab_flat.reshape(1, n)
    router_t = router_logits.transpose(0, 2, 1)
    cp, den = pl.pallas_call(
        _aux_body,
        grid=(_L,),
        in_specs=[
            pl.BlockSpec((1, _E, _N), lambda l: (l, 0, 0)),
            pl.BlockSpec((1, _N), lambda l: (0, 0)),
        ],
        out_specs=[
            pl.BlockSpec((1, 1), lambda l: (0, 0)),
            pl.BlockSpec((1, 1), lambda l: (0, 0)),
        ],
        out_shape=[
            jax.ShapeDtypeStruct((1, 1), jnp.float32),
            jax.ShapeDtypeStruct((1, 1), jnp.float32),
        ],
        scratch_shapes=[
            pltpu.VMEM((_E, 128), jnp.float32),
            pltpu.VMEM((_E, 128), jnp.float32),
            pltpu.VMEM((1, 128), jnp.float32),
        ],
        compiler_params=pltpu.CompilerParams(
            dimension_semantics=("arbitrary",)),
    )(router_t, lab2)

    g_sum = jnp.sum(g_partials)
    loss = (lse_sum[0, 0] - g_sum) / valid_cnt[0, 0]
    aux = _NUM_EXPERTS * cp[0, 0] / (den[0, 0] * den[0, 0])
    return loss + _AUX_COEF * aux
